# baseline (speedup 1.0000x reference)
import jax
import jax.numpy as jnp
from jax import lax
from jax.experimental import pallas as pl
from jax.experimental.pallas import tpu as pltpu
from jax.experimental.pallas import tpu_sc as plsc

_OUT_DIM = 49152
_VOCAB = 64
_NBLK = 8
_DC = _OUT_DIM // _NBLK
_B_TOK = 256

_NC = 2
_NS = 16
_NW = _NC * _NS
_BPW = _B_TOK // _NW


def _table_body(emb_ref, w1_ref, b1_ref, w2_ref, b2_ref, out_ref, h_ref):
    @pl.when(pl.program_id(0) == 0)
    def _():
        h_ref[...] = jnp.tanh(
            jnp.dot(emb_ref[...], w1_ref[...],
                    preferred_element_type=jnp.float32) + b1_ref[...]
        )

    out_ref[...] = (
        jnp.dot(h_ref[...], w2_ref[...], preferred_element_type=jnp.float32)
        + b2_ref[...]
    )


def _compute_table(emb, w1, b1, w2, b2):
    return pl.pallas_call(
        _table_body,
        grid=(_NBLK,),
        in_specs=[
            pl.BlockSpec((_VOCAB, 1024), lambda j: (0, 0)),
            pl.BlockSpec((1024, 512), lambda j: (0, 0)),
            pl.BlockSpec((1, 512), lambda j: (0, 0)),
            pl.BlockSpec((512, _DC), lambda j: (0, j)),
            pl.BlockSpec((1, _DC), lambda j: (0, j)),
        ],
        out_specs=pl.BlockSpec((_VOCAB, _DC), lambda j: (j, 0)),
        out_shape=jax.ShapeDtypeStruct((_NBLK * _VOCAB, _DC), jnp.float32),
        scratch_shapes=[pltpu.VMEM((_VOCAB, 512), jnp.float32)],
    )(emb, w1, b1.reshape(1, -1), w2, b2.reshape(1, -1))


def _sc_gather_body(table_hbm, idx_hbm, out_hbm,
                    idx_v, buf0, buf1, gs0, gs1, ss0, ss1):
    wid = lax.axis_index("s") * _NC + lax.axis_index("c")
    base = wid * _BPW
    pltpu.sync_copy(idx_hbm.at[pl.ds(wid * _NBLK * _BPW, _NBLK * _BPW)],
                    idx_v)

    bufs = (buf0, buf1)
    gsems = (gs0, gs1)
    ssems = (ss0, ss1)

    def idx_ref(j):
        return idx_v.at[pl.ds(j * _BPW, _BPW)]

    pltpu.async_copy(table_hbm.at[idx_ref(0)], bufs[0], gsems[0])
    for j in range(_NBLK):
        cur = j % 2
        nxt = (j + 1) % 2
        if j + 1 < _NBLK:
            if j >= 1:
                pltpu.make_async_copy(
                    bufs[nxt], out_hbm.at[pl.ds(base, _BPW), j - 1],
                    ssems[nxt]).wait()
            pltpu.async_copy(table_hbm.at[idx_ref(j + 1)], bufs[nxt],
                             gsems[nxt])
        pltpu.make_async_copy(table_hbm.at[idx_ref(j)], bufs[cur],
                              gsems[cur]).wait()
        pltpu.async_copy(bufs[cur], out_hbm.at[pl.ds(base, _BPW), j],
                         ssems[cur])
    for j in (_NBLK - 2, _NBLK - 1):
        pltpu.make_async_copy(bufs[j % 2], out_hbm.at[pl.ds(base, _BPW), j],
                              ssems[j % 2]).wait()


_sc_gather_cache = []


def _sc_gather(table2, idx2):
    if not _sc_gather_cache:
        _sc_gather_cache.append(pl.kernel(
            _sc_gather_body,
            out_type=jax.ShapeDtypeStruct((_B_TOK, _NBLK, _DC), jnp.float32),
            mesh=plsc.VectorSubcoreMesh(core_axis_name="c",
                                        subcore_axis_name="s"),
            scratch_types=[
                pltpu.VMEM((_NBLK * _BPW,), jnp.int32),
                pltpu.VMEM((_BPW, _DC), jnp.float32),
                pltpu.VMEM((_BPW, _DC), jnp.float32),
                pltpu.SemaphoreType.DMA,
                pltpu.SemaphoreType.DMA,
                pltpu.SemaphoreType.DMA,
                pltpu.SemaphoreType.DMA,
            ],
        ))
    return _sc_gather_cache[0](table2, idx2)


def kernel(prefix, emb, W1, b1, W2, b2):
    table2 = _compute_table(emb, W1, b1, W2, b2)
    pf = prefix.reshape(_NW, 1, _BPW).astype(jnp.int32)
    idx2 = (jnp.arange(_NBLK, dtype=jnp.int32)[None, :, None] * _VOCAB
            + pf).reshape(-1)
    out3 = _sc_gather(table2, idx2)
    return out3.reshape(prefix.shape[0], prefix.shape[1], _OUT_DIM)


# speedup vs baseline: 1.9928x; 1.9928x over previous
import jax
import jax.numpy as jnp
from jax import lax
from jax.experimental import pallas as pl
from jax.experimental.pallas import tpu as pltpu
from jax.experimental.pallas import tpu_sc as plsc

_LLM_DIM = 1024
_HID = 512
_OUT_DIM = 49152
_VOCAB = 64
_B_TOK = 256
_NBLK = 16
_DC = _OUT_DIM // _NBLK

_NC = 2
_NS = 16
_NW = _NC * _NS
_BPW = _B_TOK // _NW


def _emb_gather_body(emb_hbm, idx_hbm, out_hbm, idx_v, rows_v, sem):
    wid = lax.axis_index("s") * _NC + lax.axis_index("c")
    base = wid * _BPW
    pltpu.sync_copy(idx_hbm.at[pl.ds(base, _BPW)], idx_v)
    pltpu.async_copy(emb_hbm.at[idx_v], rows_v, sem).wait()
    pltpu.sync_copy(rows_v, out_hbm.at[pl.ds(base, _BPW)])


_sc_embed_cache = []


def _sc_embed(emb, idx):
    if not _sc_embed_cache:
        _sc_embed_cache.append(pl.kernel(
            _emb_gather_body,
            out_type=jax.ShapeDtypeStruct((_B_TOK, _LLM_DIM), jnp.float32),
            mesh=plsc.VectorSubcoreMesh(core_axis_name="c",
                                        subcore_axis_name="s"),
            scratch_types=[
                pltpu.VMEM((_BPW,), jnp.int32),
                pltpu.VMEM((_BPW, _LLM_DIM), jnp.float32),
                pltpu.SemaphoreType.DMA,
            ],
        ))
    return _sc_embed_cache[0](emb, idx)


def _mlp_body(x_ref, w1_ref, b1_ref, w2_ref, b2_ref, out_ref, h_ref):
    @pl.when(pl.program_id(0) == 0)
    def _():
        h = jnp.tanh(
            jnp.dot(x_ref[...], w1_ref[...],
                    preferred_element_type=jnp.float32) + b1_ref[...]
        )
        h_ref[...] = h.astype(jnp.bfloat16)

    w2b = w2_ref[...].astype(jnp.bfloat16)
    out_ref[...] = (
        jnp.dot(h_ref[...], w2b, preferred_element_type=jnp.float32)
        + b2_ref[...]
    )


def _mlp(x, w1, b1, w2, b2):
    return pl.pallas_call(
        _mlp_body,
        grid=(_NBLK,),
        in_specs=[
            pl.BlockSpec((_B_TOK, _LLM_DIM), lambda j: (0, 0)),
            pl.BlockSpec((_LLM_DIM, _HID), lambda j: (0, 0)),
            pl.BlockSpec((1, _HID), lambda j: (0, 0)),
            pl.BlockSpec((_HID, _DC), lambda j: (0, j)),
            pl.BlockSpec((1, _DC), lambda j: (0, j)),
        ],
        out_specs=pl.BlockSpec((_B_TOK, _DC), lambda j: (0, j)),
        out_shape=jax.ShapeDtypeStruct((_B_TOK, _OUT_DIM), jnp.float32),
        scratch_shapes=[pltpu.VMEM((_B_TOK, _HID), jnp.bfloat16)],
    )(x, w1, b1.reshape(1, -1), w2, b2.reshape(1, -1))


def kernel(prefix, emb, W1, b1, W2, b2):
    idx = prefix.reshape(-1).astype(jnp.int32)
    x = _sc_embed(emb, idx)
    out = _mlp(x, W1, b1, W2, b2)
    return out.reshape(prefix.shape[0], prefix.shape[1], _OUT_DIM)
